# baseline (speedup 1.0000x reference)
import functools

import jax
import jax.numpy as jnp
import numpy as np
from jax.experimental import pallas as pl

N = 50000
E = 800000
H = 64
HEADS = 4
DH = 16
NUM_LAYERS = 3

_ROWS_BLK = 512


def _combine_body(agg_ref, sk_ref, res_ref, g_ref, b_ref, o_ref, *, gelu):
    h = agg_ref[...] + sk_ref[...] + res_ref[...]
    mu = jnp.mean(h, axis=-1, keepdims=True)
    var = jnp.mean((h - mu) ** 2, axis=-1, keepdims=True)
    h = (h - mu) * jax.lax.rsqrt(var + 1e-5) * g_ref[...] + b_ref[...]
    if gelu:
        h = jax.nn.gelu(h)
    o_ref[...] = h


def _combine(agg, sk, res, g, b, gelu):
    n = agg.shape[0]
    npad = ((n + _ROWS_BLK - 1) // _ROWS_BLK) * _ROWS_BLK
    pad = npad - n
    agg = jnp.pad(agg, ((0, pad), (0, 0)))
    sk = jnp.pad(sk, ((0, pad), (0, 0)))
    res = jnp.pad(res, ((0, pad), (0, 0)))
    grid = npad // _ROWS_BLK
    blk = pl.BlockSpec((_ROWS_BLK, H), lambda i: (i, 0))
    vec = pl.BlockSpec((1, H), lambda i: (0, 0))
    out = pl.pallas_call(
        functools.partial(_combine_body, gelu=gelu),
        grid=(grid,),
        in_specs=[blk, blk, blk, vec, vec],
        out_specs=blk,
        out_shape=jax.ShapeDtypeStruct((npad, H), jnp.float32),
    )(agg, sk, res, g.reshape(1, H), b.reshape(1, H))
    return out[:n]


def _transformer_conv(x, src, dst, edge_attr, lp, n):
    q = (x @ lp['Wq'][0] + lp['Wq'][1]).reshape(n, HEADS, DH)
    k = (x @ lp['Wk'][0] + lp['Wk'][1]).reshape(n, HEADS, DH)
    v = (x @ lp['Wv'][0] + lp['Wv'][1]).reshape(n, HEADS, DH)
    e = (edge_attr @ lp['We'][0] + lp['We'][1]).reshape(-1, HEADS, DH)
    k_j = k[src] + e
    v_j = v[src] + e
    q_i = q[dst]
    score = jnp.sum(q_i * k_j, axis=-1) / jnp.sqrt(float(DH))
    m = jax.ops.segment_max(score, dst, num_segments=n)
    p = jnp.exp(score - m[dst])
    denom = jax.ops.segment_sum(p, dst, num_segments=n)
    alpha = p / (denom[dst] + 1e-16)
    agg = jax.ops.segment_sum(alpha[:, :, None] * v_j, dst, num_segments=n)
    return agg.reshape(n, HEADS * DH)


def kernel(graph_features, node_features, edge_features, edge_index, params):
    g_tok = graph_features @ params['graph_proj'][0] + params['graph_proj'][1]
    x_nodes = node_features @ params['node_proj'][0] + params['node_proj'][1]
    e_attr = edge_features @ params['edge_proj'][0] + params['edge_proj'][1]
    x = jnp.concatenate([g_tok[None, :], x_nodes], axis=0)
    n = x.shape[0]
    src = edge_index[0] + 1
    dst = edge_index[1] + 1
    node_ids = jnp.arange(1, n, dtype=src.dtype)
    zeros_ids = jnp.zeros((n - 1,), src.dtype)
    src_full = jnp.concatenate([src, zeros_ids, node_ids])
    dst_full = jnp.concatenate([dst, node_ids, zeros_ids])
    e_full = jnp.concatenate([e_attr, jnp.zeros((2 * (n - 1), H), e_attr.dtype)], axis=0)
    for i, lp in enumerate(params['layers']):
        agg = _transformer_conv(x, src_full, dst_full, e_full, lp, n)
        sk = x @ lp['Wskip'][0] + lp['Wskip'][1]
        x = _combine(agg, sk, x, lp['ln_g'], lp['ln_b'], gelu=(i < NUM_LAYERS - 1))
    return x


# speedup vs baseline: 19.2755x; 19.2755x over previous
import functools

import jax
import jax.numpy as jnp
import numpy as np
from jax import lax
from jax.experimental import pallas as pl
from jax.experimental.pallas import tpu as pltpu
from jax.experimental.pallas import tpu_sc as plsc

N = 50000
E = 800000
H = 64
HEADS = 4
DH = 16
NUM_LAYERS = 3

NP = 50176
NBLK = 98
RB = 512
EF = E + 2 * N
EP = 901120
EB = 880
C2 = 1024
GC = 256
NW = 32
EW = EP // NW
GIT = EW // GC
SBT = EB // 16
RPT = NP // 16
TRASH = N + 1

_mesh = plsc.VectorSubcoreMesh(core_axis_name="c", subcore_axis_name="s")


def _nodeproj_body(nf_ref, w_ref, b_ref, o_ref):
    o_ref[...] = jnp.dot(nf_ref[...], w_ref[...],
                         preferred_element_type=jnp.float32) + b_ref[...]


def _nodeproj(nfp, w, b):
    return pl.pallas_call(
        _nodeproj_body,
        grid=(NBLK,),
        in_specs=[pl.BlockSpec((RB, 16), lambda i: (i, 0)),
                  pl.BlockSpec((16, H), lambda i: (0, 0)),
                  pl.BlockSpec((1, H), lambda i: (0, 0))],
        out_specs=pl.BlockSpec((RB, H), lambda i: (i, 0)),
        out_shape=jax.ShapeDtypeStruct((NP, H), jnp.float32),
    )(nfp, w, b.reshape(1, H))


def _node_body(x_ref, wq_ref, bq_ref, wkv_ref, bkv_ref, ws_ref, bs_ref,
               qx_ref, kv_ref, sk_ref):
    x = x_ref[...]
    qx_ref[...] = jnp.dot(x, wq_ref[...],
                          preferred_element_type=jnp.float32) + bq_ref[...]
    kv_ref[...] = jnp.dot(x, wkv_ref[...],
                          preferred_element_type=jnp.float32) + bkv_ref[...]
    sk_ref[...] = jnp.dot(x, ws_ref[...],
                          preferred_element_type=jnp.float32) + bs_ref[...]


def _node_stage(xp, wqx, bqx, wkv, bkv, wsk, bsk):
    return pl.pallas_call(
        _node_body,
        grid=(NBLK,),
        in_specs=[pl.BlockSpec((RB, H), lambda i: (i, 0)),
                  pl.BlockSpec((H, 96), lambda i: (0, 0)),
                  pl.BlockSpec((1, 96), lambda i: (0, 0)),
                  pl.BlockSpec((H, 128), lambda i: (0, 0)),
                  pl.BlockSpec((1, 128), lambda i: (0, 0)),
                  pl.BlockSpec((H, H), lambda i: (0, 0)),
                  pl.BlockSpec((1, H), lambda i: (0, 0))],
        out_specs=[pl.BlockSpec((RB, 96), lambda i: (i, 0)),
                   pl.BlockSpec((RB, 128), lambda i: (i, 0)),
                   pl.BlockSpec((RB, H), lambda i: (i, 0))],
        out_shape=[jax.ShapeDtypeStruct((NP, 96), jnp.float32),
                   jax.ShapeDtypeStruct((NP, 128), jnp.float32),
                   jax.ShapeDtypeStruct((NP, H), jnp.float32)],
    )(xp, wqx, bqx.reshape(1, 96), wkv, bkv.reshape(1, 128),
      wsk, bsk.reshape(1, H))


def _edge_body(kvg_ref, qxg_ref, ef_ref, m5_ref, hm_ref, hm8_ref, pm_ref,
               o_ref):
    kvg = kvg_ref[...]
    qxg = qxg_ref[...]
    ef5 = ef_ref[...]
    q = qxg[:, :64]
    qm = qxg[:, 64:96]
    k = kvg[:, :64]
    v = kvg[:, 64:128]
    ef32 = jnp.concatenate([ef5, ef5, ef5, ef5], axis=1)
    s = (jnp.dot(q * k, hm_ref[...], preferred_element_type=jnp.float32)
         + jnp.dot(ef32 * qm, hm8_ref[...],
                   preferred_element_type=jnp.float32)) * 0.25
    p = jnp.exp(s)
    e = jnp.dot(ef5, m5_ref[...], preferred_element_type=jnp.float32)
    vj = v + e
    pvj = jnp.dot(p, pm_ref[...], preferred_element_type=jnp.float32) * vj
    z7 = jnp.zeros((C2, 7), jnp.float32)
    for h in range(HEADS):
        o_ref[0, h] = jnp.concatenate(
            [p[:, h:h + 1], z7, pvj[:, h * DH:(h + 1) * DH]], axis=1)


def _edge_stage(kvg, qxg, ef5, m5, hm, hm8, pm):
    return pl.pallas_call(
        _edge_body,
        grid=(EB,),
        in_specs=[pl.BlockSpec((C2, 128), lambda i: (i, 0)),
                  pl.BlockSpec((C2, 96), lambda i: (i, 0)),
                  pl.BlockSpec((C2, 8), lambda i: (i, 0)),
                  pl.BlockSpec((8, H), lambda i: (0, 0)),
                  pl.BlockSpec((H, 4), lambda i: (0, 0)),
                  pl.BlockSpec((32, 4), lambda i: (0, 0)),
                  pl.BlockSpec((4, H), lambda i: (0, 0))],
        out_specs=pl.BlockSpec((1, HEADS, C2, 24), lambda i: (i, 0, 0, 0)),
        out_shape=jax.ShapeDtypeStruct((EB, HEADS, C2, 24), jnp.float32),
    )(kvg, qxg, ef5, m5, hm, hm8, pm)


def _combine_body(a0_ref, a1_ref, a2_ref, a3_ref, sk_ref, res_ref, g_ref,
                  be_ref, o_ref, *, gelu):
    parts = []
    for r in (a0_ref, a1_ref, a2_ref, a3_ref):
        a = r[0]
        parts.append(a[:, 8:24] / (a[:, 0:1] + 1e-16))
    agg = jnp.concatenate(parts, axis=1)
    h = agg + sk_ref[...] + res_ref[...]
    mu = jnp.mean(h, axis=-1, keepdims=True)
    var = jnp.mean((h - mu) ** 2, axis=-1, keepdims=True)
    h = (h - mu) * lax.rsqrt(var + 1e-5) * g_ref[...] + be_ref[...]
    if gelu:
        h = jax.nn.gelu(h)
    o_ref[...] = h


def _combine(agg4, sk, res, g, b, gelu):
    return pl.pallas_call(
        functools.partial(_combine_body, gelu=gelu),
        grid=(NBLK,),
        in_specs=[pl.BlockSpec((1, RB, 24), lambda i: (0, i, 0)),
                  pl.BlockSpec((1, RB, 24), lambda i: (1, i, 0)),
                  pl.BlockSpec((1, RB, 24), lambda i: (2, i, 0)),
                  pl.BlockSpec((1, RB, 24), lambda i: (3, i, 0)),
                  pl.BlockSpec((RB, H), lambda i: (i, 0)),
                  pl.BlockSpec((RB, H), lambda i: (i, 0)),
                  pl.BlockSpec((1, H), lambda i: (0, 0)),
                  pl.BlockSpec((1, H), lambda i: (0, 0))],
        out_specs=pl.BlockSpec((RB, H), lambda i: (i, 0)),
        out_shape=jax.ShapeDtypeStruct((NP, H), jnp.float32),
    )(agg4, agg4, agg4, agg4, sk, res, g.reshape(1, H), b.reshape(1, H))


@functools.partial(
    pl.kernel,
    out_type=(jax.ShapeDtypeStruct((EP, 128), jnp.float32),
              jax.ShapeDtypeStruct((EP, 96), jnp.float32)),
    mesh=_mesh,
    scratch_types=[pltpu.VMEM((GC,), jnp.int32),
                   pltpu.VMEM((GC,), jnp.int32),
                   pltpu.VMEM((GC, 128), jnp.float32),
                   pltpu.VMEM((GC, 96), jnp.float32),
                   pltpu.SemaphoreType.DMA,
                   pltpu.SemaphoreType.DMA],
    compiler_params=pltpu.CompilerParams(use_tc_tiling_on_sc=False),
)
def _sc_gather(kv_hbm, qx_hbm, src_hbm, dst_hbm, kvg_hbm, qxg_hbm,
               sidx, didx, kvbuf, qxbuf, sem1, sem2):
    c = lax.axis_index("c")
    s = lax.axis_index("s")
    base = (s * 2 + c) * EW

    def body(i, carry):
        off = base + i * GC
        pltpu.sync_copy(src_hbm.at[pl.ds(off, GC)], sidx)
        pltpu.sync_copy(dst_hbm.at[pl.ds(off, GC)], didx)
        cp1 = pltpu.async_copy(kv_hbm.at[sidx], kvbuf, sem1)
        cp2 = pltpu.async_copy(qx_hbm.at[didx], qxbuf, sem2)
        cp1.wait()
        cp2.wait()
        pltpu.sync_copy(kvbuf, kvg_hbm.at[pl.ds(off, GC)])
        pltpu.sync_copy(qxbuf, qxg_hbm.at[pl.ds(off, GC)])
        return carry

    lax.fori_loop(0, GIT, body, 0)


@functools.partial(
    pl.kernel,
    out_type=jax.ShapeDtypeStruct((HEADS, NP, 24), jnp.float32),
    mesh=_mesh,
    scratch_types=[pltpu.VMEM((128, 24), jnp.float32),
                   pltpu.VMEM((128,), jnp.int32),
                   pltpu.VMEM_SHARED((NP, 24), jnp.float32)],
    compiler_params=pltpu.CompilerParams(use_tc_tiling_on_sc=False),
)
def _sc_scatter(pv_hbm, dst3_hbm, zeros_hbm, out_hbm, pvbuf, dibuf, acc):
    c = lax.axis_index("c")
    s = lax.axis_index("s")
    for ph in range(2):
        a = c * 2 + ph
        pltpu.sync_copy(zeros_hbm, acc.at[pl.ds(s * RPT, RPT)])
        plsc.subcore_barrier()

        def body(t, carry):
            b = s + 16 * (t // 8)
            j = t % 8
            pltpu.sync_copy(pv_hbm.at[b, a, pl.ds(j * 128, 128)], pvbuf)
            pltpu.sync_copy(dst3_hbm.at[b, j], dibuf)
            pltpu.sync_copy(pvbuf, acc.at[dibuf], add=True)
            return carry

        lax.fori_loop(0, SBT * 8, body, 0)
        plsc.subcore_barrier()
        pltpu.sync_copy(acc.at[pl.ds(s * RPT, RPT)],
                        out_hbm.at[a, pl.ds(s * RPT, RPT)])
        plsc.subcore_barrier()


def _fold_weights(params):
    wep, bep = params['edge_proj']
    folded = []
    for lp in params['layers']:
        wq, bq = lp['Wq']
        wk, bk = lp['Wk']
        wv, bv = lp['Wv']
        we, bwe = lp['We']
        ws, bs = lp['Wskip']
        m = wep @ we
        cl = bep @ we + bwe
        m5 = jnp.concatenate([m, cl[None, :], bwe[None, :],
                              jnp.zeros((3, H), jnp.float32)], axis=0)
        t = jnp.zeros((H, 32), jnp.float32)
        for h in range(HEADS):
            t = t.at[h * DH:(h + 1) * DH, h * 8:h * 8 + 8].set(
                m5[:, h * DH:(h + 1) * DH].T)
        wqx = jnp.concatenate([wq, wq @ t], axis=1)
        bqx = jnp.concatenate([bq, bq @ t], axis=0)
        wkv = jnp.concatenate([wk, wv], axis=1)
        bkv = jnp.concatenate([bk, bv], axis=0)
        folded.append(dict(m5=m5, wqx=wqx, bqx=bqx, wkv=wkv, bkv=bkv,
                           wsk=ws, bsk=bs, g=lp['ln_g'], b=lp['ln_b']))
    return folded


def kernel(graph_features, node_features, edge_features, edge_index, params):
    f32 = jnp.float32
    src = edge_index[0] + 1
    dst = edge_index[1] + 1
    node_ids = jnp.arange(1, N + 1, dtype=jnp.int32)
    zeros_ids = jnp.zeros((N,), jnp.int32)
    src_full = jnp.concatenate([src, zeros_ids, node_ids])
    dst_full = jnp.concatenate([dst, node_ids, zeros_ids])
    pad = EP - EF
    src_g = jnp.concatenate([src_full, jnp.zeros((pad,), jnp.int32)])
    dst_g = jnp.concatenate([dst_full, jnp.zeros((pad,), jnp.int32)])
    dst_s = jnp.concatenate(
        [dst_full, jnp.full((pad,), TRASH, jnp.int32)]).reshape(EB, 8, 128)

    ef5 = jnp.concatenate(
        [jnp.concatenate([edge_features,
                          jnp.ones((E, 1), f32),
                          jnp.zeros((E, 4), f32)], axis=1),
         jnp.tile(jnp.array([[0, 0, 0, 0, 1, 0, 0, 0]], f32), (2 * N, 1)),
         jnp.zeros((pad, 8), f32)], axis=0)

    hm = np.zeros((H, 4), np.float32)
    for h in range(HEADS):
        hm[h * DH:(h + 1) * DH, h] = 1.0
    hm8 = np.zeros((32, 4), np.float32)
    for h in range(HEADS):
        hm8[h * 8:h * 8 + 8, h] = 1.0
    hm = jnp.asarray(hm)
    hm8 = jnp.asarray(hm8)
    pm = jnp.asarray(hm.T)
    zrows = jnp.zeros((RPT, 24), f32)
    folded = _fold_weights(params)

    g_tok = graph_features @ params['graph_proj'][0] + params['graph_proj'][1]
    nfp = jnp.zeros((NP, 16), f32).at[:N, :9].set(node_features)
    xn = _nodeproj(nfp, jnp.zeros((16, H), f32).at[:9].set(
        params['node_proj'][0]), params['node_proj'][1])
    xp = jnp.concatenate([g_tok[None, :], xn[:NP - 1]], axis=0)

    for i, fw in enumerate(folded):
        qx, kv, sk = _node_stage(xp, fw['wqx'], fw['bqx'], fw['wkv'],
                                 fw['bkv'], fw['wsk'], fw['bsk'])
        kvg, qxg = _sc_gather(kv, qx, src_g, dst_g)
        pv = _edge_stage(kvg, qxg, ef5, fw['m5'], hm, hm8, pm)
        agg4 = _sc_scatter(pv, dst_s, zrows)
        xp = _combine(agg4, sk, xp, fw['g'], fw['b'],
                      gelu=(i < NUM_LAYERS - 1))
    return xp[:N + 1]


# speedup vs baseline: 20.7420x; 1.0761x over previous
import functools

import jax
import jax.numpy as jnp
import numpy as np
from jax import lax
from jax.experimental import pallas as pl
from jax.experimental.pallas import tpu as pltpu
from jax.experimental.pallas import tpu_sc as plsc

N = 50000
E = 800000
H = 64
HEADS = 4
DH = 16
NUM_LAYERS = 3

NP = 50176
NBLK = 98
RB = 512
EF = E + 2 * N
EP = 901120
EB = 440
C2 = 2048
CHQ = EP // 128
CPT = CHQ // 16
GC = 128
NW = 32
EW = EP // NW
GIT = EW // GC
RPT = NP // 16
TRASH = N + 1

_mesh = plsc.VectorSubcoreMesh(core_axis_name="c", subcore_axis_name="s")


def _nodeproj_body(nf_ref, w_ref, b_ref, o_ref):
    o_ref[...] = jnp.dot(nf_ref[...], w_ref[...],
                         preferred_element_type=jnp.float32) + b_ref[...]


def _nodeproj(nfp, w, b):
    return pl.pallas_call(
        _nodeproj_body,
        grid=(NBLK,),
        in_specs=[pl.BlockSpec((RB, 16), lambda i: (i, 0)),
                  pl.BlockSpec((16, H), lambda i: (0, 0)),
                  pl.BlockSpec((1, H), lambda i: (0, 0))],
        out_specs=pl.BlockSpec((RB, H), lambda i: (i, 0)),
        out_shape=jax.ShapeDtypeStruct((NP, H), jnp.float32),
    )(nfp, w, b.reshape(1, H))


def _node_body(x_ref, wq_ref, bq_ref, wkv_ref, bkv_ref, ws_ref, bs_ref,
               qx_ref, kv_ref, sk_ref):
    x = x_ref[...]
    qx_ref[...] = jnp.dot(x, wq_ref[...],
                          preferred_element_type=jnp.float32) + bq_ref[...]
    kv_ref[...] = jnp.dot(x, wkv_ref[...],
                          preferred_element_type=jnp.float32) + bkv_ref[...]
    sk_ref[...] = jnp.dot(x, ws_ref[...],
                          preferred_element_type=jnp.float32) + bs_ref[...]


def _node_stage(xp, wqx, bqx, wkv, bkv, wsk, bsk):
    return pl.pallas_call(
        _node_body,
        grid=(NBLK,),
        in_specs=[pl.BlockSpec((RB, H), lambda i: (i, 0)),
                  pl.BlockSpec((H, 128), lambda i: (0, 0)),
                  pl.BlockSpec((1, 128), lambda i: (0, 0)),
                  pl.BlockSpec((H, 128), lambda i: (0, 0)),
                  pl.BlockSpec((1, 128), lambda i: (0, 0)),
                  pl.BlockSpec((H, H), lambda i: (0, 0)),
                  pl.BlockSpec((1, H), lambda i: (0, 0))],
        out_specs=[pl.BlockSpec((RB, 128), lambda i: (i, 0)),
                   pl.BlockSpec((RB, 128), lambda i: (i, 0)),
                   pl.BlockSpec((RB, H), lambda i: (i, 0))],
        out_shape=[jax.ShapeDtypeStruct((NP, 128), jnp.float32),
                   jax.ShapeDtypeStruct((NP, 128), jnp.float32),
                   jax.ShapeDtypeStruct((NP, H), jnp.float32)],
    )(xp, wqx, bqx.reshape(1, 128), wkv, bkv.reshape(1, 128),
      wsk, bsk.reshape(1, H))


def _edge_body(kvg_ref, qxg_ref, ef_ref, m5_ref, hm_ref, hm8_ref, pm_ref,
               o_ref):
    kvg = kvg_ref[...]
    qxg = qxg_ref[...]
    ef5 = ef_ref[...]
    q = qxg[:, :64]
    qm = qxg[:, 64:96]
    k = kvg[:, :64]
    v = kvg[:, 64:128]
    ef32 = jnp.concatenate([ef5, ef5, ef5, ef5], axis=1)
    s = (jnp.dot(q * k, hm_ref[...], preferred_element_type=jnp.float32)
         + jnp.dot(ef32 * qm, hm8_ref[...],
                   preferred_element_type=jnp.float32)) * 0.25
    p = jnp.exp(s)
    e = jnp.dot(ef5, m5_ref[...], preferred_element_type=jnp.float32)
    vj = v + e
    pvj = jnp.dot(p, pm_ref[...], preferred_element_type=jnp.float32) * vj
    z7 = jnp.zeros((C2, 7), jnp.float32)
    for h in range(HEADS):
        o_ref[0, h] = jnp.concatenate(
            [p[:, h:h + 1], z7, pvj[:, h * DH:(h + 1) * DH]], axis=1)


def _edge_stage(kvg, qxg, ef5, m5, hm, hm8, pm):
    return pl.pallas_call(
        _edge_body,
        grid=(EB,),
        in_specs=[pl.BlockSpec((C2, 128), lambda i: (i, 0)),
                  pl.BlockSpec((C2, 128), lambda i: (i, 0)),
                  pl.BlockSpec((C2, 8), lambda i: (i, 0)),
                  pl.BlockSpec((8, H), lambda i: (0, 0)),
                  pl.BlockSpec((H, 4), lambda i: (0, 0)),
                  pl.BlockSpec((32, 4), lambda i: (0, 0)),
                  pl.BlockSpec((4, H), lambda i: (0, 0))],
        out_specs=pl.BlockSpec((1, HEADS, C2, 24), lambda i: (i, 0, 0, 0)),
        out_shape=jax.ShapeDtypeStruct((EB, HEADS, C2, 24), jnp.float32),
    )(kvg, qxg, ef5, m5, hm, hm8, pm)


def _combine_body(a0_ref, a1_ref, a2_ref, a3_ref, sk_ref, res_ref, g_ref,
                  be_ref, o_ref, *, gelu):
    parts = []
    for r in (a0_ref, a1_ref, a2_ref, a3_ref):
        a = r[0]
        parts.append(a[:, 8:24] / (a[:, 0:1] + 1e-16))
    agg = jnp.concatenate(parts, axis=1)
    h = agg + sk_ref[...] + res_ref[...]
    mu = jnp.mean(h, axis=-1, keepdims=True)
    var = jnp.mean((h - mu) ** 2, axis=-1, keepdims=True)
    h = (h - mu) * lax.rsqrt(var + 1e-5) * g_ref[...] + be_ref[...]
    if gelu:
        h = jax.nn.gelu(h)
    o_ref[...] = h


def _combine(agg4, sk, res, g, b, gelu):
    return pl.pallas_call(
        functools.partial(_combine_body, gelu=gelu),
        grid=(NBLK,),
        in_specs=[pl.BlockSpec((1, RB, 24), lambda i: (0, i, 0)),
                  pl.BlockSpec((1, RB, 24), lambda i: (1, i, 0)),
                  pl.BlockSpec((1, RB, 24), lambda i: (2, i, 0)),
                  pl.BlockSpec((1, RB, 24), lambda i: (3, i, 0)),
                  pl.BlockSpec((RB, H), lambda i: (i, 0)),
                  pl.BlockSpec((RB, H), lambda i: (i, 0)),
                  pl.BlockSpec((1, H), lambda i: (0, 0)),
                  pl.BlockSpec((1, H), lambda i: (0, 0))],
        out_specs=pl.BlockSpec((RB, H), lambda i: (i, 0)),
        out_shape=jax.ShapeDtypeStruct((NP, H), jnp.float32),
    )(agg4, agg4, agg4, agg4, sk, res, g.reshape(1, H), b.reshape(1, H))


@functools.partial(
    pl.kernel,
    out_type=(jax.ShapeDtypeStruct((EP, 128), jnp.float32),
              jax.ShapeDtypeStruct((EP, 128), jnp.float32)),
    mesh=_mesh,
    scratch_types=[pltpu.VMEM((GC,), jnp.int32),
                   pltpu.VMEM((GC,), jnp.int32),
                   pltpu.VMEM((GC,), jnp.int32),
                   pltpu.VMEM((GC,), jnp.int32),
                   pltpu.VMEM((2, GC, 128), jnp.float32),
                   pltpu.VMEM((2, GC, 128), jnp.float32),
                   pltpu.SemaphoreType.DMA,
                   pltpu.SemaphoreType.DMA,
                   pltpu.SemaphoreType.DMA,
                   pltpu.SemaphoreType.DMA,
                   pltpu.SemaphoreType.DMA,
                   pltpu.SemaphoreType.DMA,
                   pltpu.SemaphoreType.DMA,
                   pltpu.SemaphoreType.DMA],
)
def _sc_gather(kv_hbm, qx_hbm, sdc_hbm, kvg_hbm, qxg_hbm,
               si0, si1, di0, di1, kvb, qxb, gkv0, gkv1, gqx0, gqx1,
               wkv0, wkv1, wqx0, wqx1):
    c = lax.axis_index("c")
    s = lax.axis_index("s")
    w = s * 2 + c
    base = w * EW
    si = (si0, si1)
    di = (di0, di1)
    gkv = (gkv0, gkv1)
    gqx = (gqx0, gqx1)
    wkv = (wkv0, wkv1)
    wqx = (wqx0, wqx1)

    def do_iter(i, b, first):
        if not first:
            poff = base + (i - 2) * GC
            pltpu.make_async_copy(kvb.at[b], kvg_hbm.at[pl.ds(poff, GC)],
                                  wkv[b]).wait()
            pltpu.make_async_copy(qxb.at[b], qxg_hbm.at[pl.ds(poff, GC)],
                                  wqx[b]).wait()
        pltpu.sync_copy(sdc_hbm.at[w, i, 0], si[b])
        pltpu.sync_copy(sdc_hbm.at[w, i, 1], di[b])
        cpk = pltpu.async_copy(kv_hbm.at[si[b]], kvb.at[b], gkv[b])
        cpq = pltpu.async_copy(qx_hbm.at[di[b]], qxb.at[b], gqx[b])
        cpk.wait()
        cpq.wait()
        off = base + i * GC
        pltpu.async_copy(kvb.at[b], kvg_hbm.at[pl.ds(off, GC)], wkv[b])
        pltpu.async_copy(qxb.at[b], qxg_hbm.at[pl.ds(off, GC)], wqx[b])

    do_iter(0, 0, True)
    do_iter(1, 1, True)

    def body(t, carry):
        do_iter(2 * t, 0, False)
        do_iter(2 * t + 1, 1, False)
        return carry

    lax.fori_loop(1, GIT // 2, body, 0)
    foff = base + (GIT - 2) * GC
    pltpu.make_async_copy(kvb.at[0], kvg_hbm.at[pl.ds(foff, GC)],
                          wkv[0]).wait()
    pltpu.make_async_copy(qxb.at[0], qxg_hbm.at[pl.ds(foff, GC)],
                          wqx[0]).wait()
    foff = base + (GIT - 1) * GC
    pltpu.make_async_copy(kvb.at[1], kvg_hbm.at[pl.ds(foff, GC)],
                          wkv[1]).wait()
    pltpu.make_async_copy(qxb.at[1], qxg_hbm.at[pl.ds(foff, GC)],
                          wqx[1]).wait()


@functools.partial(
    pl.kernel,
    out_type=jax.ShapeDtypeStruct((HEADS, NP, 24), jnp.float32),
    mesh=_mesh,
    scratch_types=[pltpu.VMEM((128, 24), jnp.float32),
                   pltpu.VMEM((128,), jnp.int32),
                   pltpu.VMEM_SHARED((NP, 24), jnp.float32)],
    compiler_params=pltpu.CompilerParams(use_tc_tiling_on_sc=False),
)
def _sc_scatter(pv_hbm, dst2_hbm, zeros_hbm, out_hbm, pvb, dib, acc):
    c = lax.axis_index("c")
    s = lax.axis_index("s")
    nb = C2 // 128

    for ph in range(2):
        a = c * 2 + ph
        pltpu.sync_copy(zeros_hbm, acc.at[pl.ds(s * RPT, RPT)])
        plsc.subcore_barrier()

        def body(i, carry):
            q = s + 16 * i
            blk = q // nb
            j = q % nb
            pltpu.sync_copy(pv_hbm.at[blk, a, pl.ds(j * 128, 128)], pvb)
            pltpu.sync_copy(dst2_hbm.at[q], dib)
            pltpu.sync_copy(pvb, acc.at[dib], add=True)
            return carry

        lax.fori_loop(0, CPT, body, 0)
        plsc.subcore_barrier()
        pltpu.sync_copy(acc.at[pl.ds(s * RPT, RPT)],
                        out_hbm.at[a, pl.ds(s * RPT, RPT)])
        plsc.subcore_barrier()


def _fold_weights(params):
    wep, bep = params['edge_proj']
    folded = []
    for lp in params['layers']:
        wq, bq = lp['Wq']
        wk, bk = lp['Wk']
        wv, bv = lp['Wv']
        we, bwe = lp['We']
        ws, bs = lp['Wskip']
        m = wep @ we
        cl = bep @ we + bwe
        m5 = jnp.concatenate([m, cl[None, :], bwe[None, :],
                              jnp.zeros((3, H), jnp.float32)], axis=0)
        t = jnp.zeros((H, 32), jnp.float32)
        for h in range(HEADS):
            t = t.at[h * DH:(h + 1) * DH, h * 8:h * 8 + 8].set(
                m5[:, h * DH:(h + 1) * DH].T)
        wqx = jnp.concatenate([wq, wq @ t, jnp.zeros((H, 32), jnp.float32)],
                              axis=1)
        bqx = jnp.concatenate([bq, bq @ t, jnp.zeros((32,), jnp.float32)],
                              axis=0)
        wkv = jnp.concatenate([wk, wv], axis=1)
        bkv = jnp.concatenate([bk, bv], axis=0)
        folded.append(dict(m5=m5, wqx=wqx, bqx=bqx, wkv=wkv, bkv=bkv,
                           wsk=ws, bsk=bs, g=lp['ln_g'], b=lp['ln_b']))
    return folded


def kernel(graph_features, node_features, edge_features, edge_index, params):
    f32 = jnp.float32
    src = edge_index[0] + 1
    dst = edge_index[1] + 1
    node_ids = jnp.arange(1, N + 1, dtype=jnp.int32)
    zeros_ids = jnp.zeros((N,), jnp.int32)
    src_full = jnp.concatenate([src, zeros_ids, node_ids])
    dst_full = jnp.concatenate([dst, node_ids, zeros_ids])
    pad = EP - EF
    src_g = jnp.concatenate([src_full, jnp.zeros((pad,), jnp.int32)])
    dst_g = jnp.concatenate([dst_full, jnp.zeros((pad,), jnp.int32)])
    dst_s = jnp.concatenate(
        [dst_full, jnp.full((pad,), TRASH, jnp.int32)]).reshape(CHQ, 128)
    sdc = jnp.stack([src_g.reshape(NW, GIT, GC),
                     dst_g.reshape(NW, GIT, GC)], axis=2)

    ef5 = jnp.concatenate(
        [jnp.concatenate([edge_features,
                          jnp.ones((E, 1), f32),
                          jnp.zeros((E, 4), f32)], axis=1),
         jnp.tile(jnp.array([[0, 0, 0, 0, 1, 0, 0, 0]], f32), (2 * N, 1)),
         jnp.zeros((pad, 8), f32)], axis=0)

    hm = np.zeros((H, 4), np.float32)
    for h in range(HEADS):
        hm[h * DH:(h + 1) * DH, h] = 1.0
    hm8 = np.zeros((32, 4), np.float32)
    for h in range(HEADS):
        hm8[h * 8:h * 8 + 8, h] = 1.0
    hm = jnp.asarray(hm)
    hm8 = jnp.asarray(hm8)
    pm = jnp.asarray(hm.T)
    zrows = jnp.zeros((RPT, 24), f32)
    folded = _fold_weights(params)

    g_tok = graph_features @ params['graph_proj'][0] + params['graph_proj'][1]
    nfp = jnp.zeros((NP, 16), f32).at[:N, :9].set(node_features)
    xn = _nodeproj(nfp, jnp.zeros((16, H), f32).at[:9].set(
        params['node_proj'][0]), params['node_proj'][1])
    xp = jnp.concatenate([g_tok[None, :], xn[:NP - 1]], axis=0)

    for i, fw in enumerate(folded):
        qx, kv, sk = _node_stage(xp, fw['wqx'], fw['bqx'], fw['wkv'],
                                 fw['bkv'], fw['wsk'], fw['bsk'])
        kvg, qxg = _sc_gather(kv, qx, sdc)
        pv = _edge_stage(kvg, qxg, ef5, fw['m5'], hm, hm8, pm)
        agg4 = _sc_scatter(pv, dst_s, zrows)
        xp = _combine(agg4, sk, xp, fw['g'], fw['b'],
                      gelu=(i < NUM_LAYERS - 1))
    return xp[:N + 1]


# speedup vs baseline: 22.2700x; 1.0737x over previous
import functools

import jax
import jax.numpy as jnp
import numpy as np
from jax import lax
from jax.experimental import pallas as pl
from jax.experimental.pallas import tpu as pltpu
from jax.experimental.pallas import tpu_sc as plsc

N = 50000
E = 800000
H = 64
HEADS = 4
DH = 16
NUM_LAYERS = 3

NP = 50176
NBLK = 98
RB = 512
EF = E + 2 * N
EP = 901120
EB = 440
C2 = 2048
CHQ = EP // 128
CPT = CHQ // 16
GC = 128
NW = 32
EW = EP // NW
GIT = EW // GC
RPT = NP // 16
TRASH = N + 1

_mesh = plsc.VectorSubcoreMesh(core_axis_name="c", subcore_axis_name="s")


def _nodeproj_body(nf_ref, w_ref, b_ref, o_ref):
    o_ref[...] = jnp.dot(nf_ref[...], w_ref[...],
                         preferred_element_type=jnp.float32) + b_ref[...]


def _nodeproj(nfp, w, b):
    return pl.pallas_call(
        _nodeproj_body,
        grid=(NBLK,),
        in_specs=[pl.BlockSpec((RB, 16), lambda i: (i, 0)),
                  pl.BlockSpec((16, H), lambda i: (0, 0)),
                  pl.BlockSpec((1, H), lambda i: (0, 0))],
        out_specs=pl.BlockSpec((RB, H), lambda i: (i, 0)),
        out_shape=jax.ShapeDtypeStruct((NP, H), jnp.float32),
    )(nfp, w, b.reshape(1, H))


def _node_body(x_ref, wq_ref, bq_ref, wkv_ref, bkv_ref, ws_ref, bs_ref,
               qx_ref, kv_ref, sk_ref):
    x = x_ref[...]
    qx_ref[...] = jnp.dot(x, wq_ref[...],
                          preferred_element_type=jnp.float32) + bq_ref[...]
    kv_ref[...] = jnp.dot(x, wkv_ref[...],
                          preferred_element_type=jnp.float32) + bkv_ref[...]
    sk_ref[...] = jnp.dot(x, ws_ref[...],
                          preferred_element_type=jnp.float32) + bs_ref[...]


def _node_stage(xp, wqx, bqx, wkv, bkv, wsk, bsk):
    return pl.pallas_call(
        _node_body,
        grid=(NBLK,),
        in_specs=[pl.BlockSpec((RB, H), lambda i: (i, 0)),
                  pl.BlockSpec((H, 128), lambda i: (0, 0)),
                  pl.BlockSpec((1, 128), lambda i: (0, 0)),
                  pl.BlockSpec((H, 128), lambda i: (0, 0)),
                  pl.BlockSpec((1, 128), lambda i: (0, 0)),
                  pl.BlockSpec((H, H), lambda i: (0, 0)),
                  pl.BlockSpec((1, H), lambda i: (0, 0))],
        out_specs=[pl.BlockSpec((RB, 128), lambda i: (i, 0)),
                   pl.BlockSpec((RB, 128), lambda i: (i, 0)),
                   pl.BlockSpec((RB, H), lambda i: (i, 0))],
        out_shape=[jax.ShapeDtypeStruct((NP, 128), jnp.float32),
                   jax.ShapeDtypeStruct((NP, 128), jnp.float32),
                   jax.ShapeDtypeStruct((NP, H), jnp.float32)],
    )(xp, wqx, bqx.reshape(1, 128), wkv, bkv.reshape(1, 128),
      wsk, bsk.reshape(1, H))


def _edge_body(kvg_ref, qxg_ref, ef_ref, m5_ref, hm_ref, hm8_ref, pm_ref,
               o_ref):
    kvg = kvg_ref[...]
    qxg = qxg_ref[...]
    ef5 = ef_ref[...]
    q = qxg[:, :64]
    qm = qxg[:, 64:96]
    k = kvg[:, :64]
    v = kvg[:, 64:128]
    ef32 = jnp.concatenate([ef5, ef5, ef5, ef5], axis=1)
    s = (jnp.dot(q * k, hm_ref[...], preferred_element_type=jnp.float32)
         + jnp.dot(ef32 * qm, hm8_ref[...],
                   preferred_element_type=jnp.float32)) * 0.25
    p = jnp.exp(s)
    e = jnp.dot(ef5, m5_ref[...], preferred_element_type=jnp.float32)
    vj = v + e
    pvj = jnp.dot(p, pm_ref[...], preferred_element_type=jnp.float32) * vj
    z7 = jnp.zeros((C2, 7), jnp.float32)
    for h in range(HEADS):
        o_ref[0, h] = jnp.concatenate(
            [p[:, h:h + 1], z7, pvj[:, h * DH:(h + 1) * DH]], axis=1)


def _edge_stage(kvg, qxg, ef5, m5, hm, hm8, pm):
    return pl.pallas_call(
        _edge_body,
        grid=(EB,),
        in_specs=[pl.BlockSpec((C2, 128), lambda i: (i, 0)),
                  pl.BlockSpec((C2, 128), lambda i: (i, 0)),
                  pl.BlockSpec((C2, 8), lambda i: (i, 0)),
                  pl.BlockSpec((8, H), lambda i: (0, 0)),
                  pl.BlockSpec((H, 4), lambda i: (0, 0)),
                  pl.BlockSpec((32, 4), lambda i: (0, 0)),
                  pl.BlockSpec((4, H), lambda i: (0, 0))],
        out_specs=pl.BlockSpec((1, HEADS, C2, 24), lambda i: (i, 0, 0, 0)),
        out_shape=jax.ShapeDtypeStruct((EB, HEADS, C2, 24), jnp.float32),
    )(kvg, qxg, ef5, m5, hm, hm8, pm)


def _combine_body(a0_ref, a1_ref, a2_ref, a3_ref, sk_ref, res_ref, g_ref,
                  be_ref, o_ref, *, gelu):
    parts = []
    for r in (a0_ref, a1_ref, a2_ref, a3_ref):
        a = r[0]
        parts.append(a[:, 8:24] / (a[:, 0:1] + 1e-16))
    agg = jnp.concatenate(parts, axis=1)
    h = agg + sk_ref[...] + res_ref[...]
    mu = jnp.mean(h, axis=-1, keepdims=True)
    var = jnp.mean((h - mu) ** 2, axis=-1, keepdims=True)
    h = (h - mu) * lax.rsqrt(var + 1e-5) * g_ref[...] + be_ref[...]
    if gelu:
        h = jax.nn.gelu(h)
    o_ref[...] = h


def _combine(agg4, sk, res, g, b, gelu):
    return pl.pallas_call(
        functools.partial(_combine_body, gelu=gelu),
        grid=(NBLK,),
        in_specs=[pl.BlockSpec((1, RB, 24), lambda i: (0, i, 0)),
                  pl.BlockSpec((1, RB, 24), lambda i: (1, i, 0)),
                  pl.BlockSpec((1, RB, 24), lambda i: (2, i, 0)),
                  pl.BlockSpec((1, RB, 24), lambda i: (3, i, 0)),
                  pl.BlockSpec((RB, H), lambda i: (i, 0)),
                  pl.BlockSpec((RB, H), lambda i: (i, 0)),
                  pl.BlockSpec((1, H), lambda i: (0, 0)),
                  pl.BlockSpec((1, H), lambda i: (0, 0))],
        out_specs=pl.BlockSpec((RB, H), lambda i: (i, 0)),
        out_shape=jax.ShapeDtypeStruct((NP, H), jnp.float32),
    )(agg4, agg4, agg4, agg4, sk, res, g.reshape(1, H), b.reshape(1, H))


@functools.partial(
    pl.kernel,
    out_type=(jax.ShapeDtypeStruct((EP, 128), jnp.float32),
              jax.ShapeDtypeStruct((EP, 128), jnp.float32)),
    mesh=_mesh,
    scratch_types=[pltpu.VMEM((GC,), jnp.int32),
                   pltpu.VMEM((GC,), jnp.int32),
                   pltpu.VMEM((GC,), jnp.int32),
                   pltpu.VMEM((GC,), jnp.int32),
                   pltpu.VMEM((2, GC, 128), jnp.float32),
                   pltpu.VMEM((2, GC, 128), jnp.float32),
                   pltpu.SemaphoreType.DMA,
                   pltpu.SemaphoreType.DMA,
                   pltpu.SemaphoreType.DMA,
                   pltpu.SemaphoreType.DMA,
                   pltpu.SemaphoreType.DMA,
                   pltpu.SemaphoreType.DMA,
                   pltpu.SemaphoreType.DMA,
                   pltpu.SemaphoreType.DMA],
)
def _sc_gather(kv_hbm, qx_hbm, sdc_hbm, kvg_hbm, qxg_hbm,
               si0, si1, di0, di1, kvb, qxb, gkv0, gkv1, gqx0, gqx1,
               wkv0, wkv1, wqx0, wqx1):
    c = lax.axis_index("c")
    s = lax.axis_index("s")
    w = s * 2 + c
    base = w * EW
    si = (si0, si1)
    di = (di0, di1)
    gkv = (gkv0, gkv1)
    gqx = (gqx0, gqx1)
    wkv = (wkv0, wkv1)
    wqx = (wqx0, wqx1)

    def do_iter(i, b, first):
        if not first:
            poff = base + (i - 2) * GC
            pltpu.make_async_copy(kvb.at[b], kvg_hbm.at[pl.ds(poff, GC)],
                                  wkv[b]).wait()
            pltpu.make_async_copy(qxb.at[b], qxg_hbm.at[pl.ds(poff, GC)],
                                  wqx[b]).wait()
        pltpu.sync_copy(sdc_hbm.at[w, i, 0], si[b])
        pltpu.sync_copy(sdc_hbm.at[w, i, 1], di[b])
        cpk = pltpu.async_copy(kv_hbm.at[si[b]], kvb.at[b], gkv[b])
        cpq = pltpu.async_copy(qx_hbm.at[di[b]], qxb.at[b], gqx[b])
        cpk.wait()
        cpq.wait()
        off = base + i * GC
        pltpu.async_copy(kvb.at[b], kvg_hbm.at[pl.ds(off, GC)], wkv[b])
        pltpu.async_copy(qxb.at[b], qxg_hbm.at[pl.ds(off, GC)], wqx[b])

    do_iter(0, 0, True)
    do_iter(1, 1, True)

    def body(t, carry):
        do_iter(2 * t, 0, False)
        do_iter(2 * t + 1, 1, False)
        return carry

    lax.fori_loop(1, GIT // 2, body, 0)
    foff = base + (GIT - 2) * GC
    pltpu.make_async_copy(kvb.at[0], kvg_hbm.at[pl.ds(foff, GC)],
                          wkv[0]).wait()
    pltpu.make_async_copy(qxb.at[0], qxg_hbm.at[pl.ds(foff, GC)],
                          wqx[0]).wait()
    foff = base + (GIT - 1) * GC
    pltpu.make_async_copy(kvb.at[1], kvg_hbm.at[pl.ds(foff, GC)],
                          wkv[1]).wait()
    pltpu.make_async_copy(qxb.at[1], qxg_hbm.at[pl.ds(foff, GC)],
                          wqx[1]).wait()


@functools.partial(
    pl.kernel,
    out_type=jax.ShapeDtypeStruct((HEADS, NP, 24), jnp.float32),
    mesh=_mesh,
    scratch_types=[pltpu.VMEM((2, 128, 24), jnp.float32),
                   pltpu.VMEM((128,), jnp.int32),
                   pltpu.VMEM_SHARED((NP, 24), jnp.float32),
                   pltpu.SemaphoreType.DMA,
                   pltpu.SemaphoreType.DMA],
    compiler_params=pltpu.CompilerParams(use_tc_tiling_on_sc=False),
)
def _sc_scatter(pv_hbm, dst2_hbm, zeros_hbm, out_hbm, pvb, dib, acc,
                spv0, spv1):
    c = lax.axis_index("c")
    s = lax.axis_index("s")
    nb = C2 // 128
    spv = (spv0, spv1)

    for ph in range(2):
        a = c * 2 + ph

        def pv_src(i):
            q = s + 16 * i
            return pv_hbm.at[q // nb, a, pl.ds((q % nb) * 128, 128)]

        pltpu.sync_copy(zeros_hbm, acc.at[pl.ds(s * RPT, RPT)])
        plsc.subcore_barrier()
        pltpu.async_copy(pv_src(0), pvb.at[0], spv[0])
        pltpu.async_copy(pv_src(1), pvb.at[1], spv[1])

        def step(i, b, last):
            pltpu.make_async_copy(pv_src(i), pvb.at[b], spv[b]).wait()
            pltpu.sync_copy(dst2_hbm.at[s + 16 * i], dib)
            pltpu.sync_copy(pvb.at[b], acc.at[dib], add=True)
            if not last:
                pltpu.async_copy(pv_src(i + 2), pvb.at[b], spv[b])

        def body(t, carry):
            step(2 * t, 0, False)
            step(2 * t + 1, 1, False)
            return carry

        lax.fori_loop(0, CPT // 2 - 1, body, 0)
        step(CPT - 2, 0, True)
        step(CPT - 1, 1, True)
        plsc.subcore_barrier()
        pltpu.sync_copy(acc.at[pl.ds(s * RPT, RPT)],
                        out_hbm.at[a, pl.ds(s * RPT, RPT)])
        plsc.subcore_barrier()


def _fold_weights(params):
    wep, bep = params['edge_proj']
    folded = []
    for lp in params['layers']:
        wq, bq = lp['Wq']
        wk, bk = lp['Wk']
        wv, bv = lp['Wv']
        we, bwe = lp['We']
        ws, bs = lp['Wskip']
        m = wep @ we
        cl = bep @ we + bwe
        m5 = jnp.concatenate([m, cl[None, :], bwe[None, :],
                              jnp.zeros((3, H), jnp.float32)], axis=0)
        t = jnp.zeros((H, 32), jnp.float32)
        for h in range(HEADS):
            t = t.at[h * DH:(h + 1) * DH, h * 8:h * 8 + 8].set(
                m5[:, h * DH:(h + 1) * DH].T)
        wqx = jnp.concatenate([wq, wq @ t, jnp.zeros((H, 32), jnp.float32)],
                              axis=1)
        bqx = jnp.concatenate([bq, bq @ t, jnp.zeros((32,), jnp.float32)],
                              axis=0)
        wkv = jnp.concatenate([wk, wv], axis=1)
        bkv = jnp.concatenate([bk, bv], axis=0)
        folded.append(dict(m5=m5, wqx=wqx, bqx=bqx, wkv=wkv, bkv=bkv,
                           wsk=ws, bsk=bs, g=lp['ln_g'], b=lp['ln_b']))
    return folded


def kernel(graph_features, node_features, edge_features, edge_index, params):
    f32 = jnp.float32
    src = edge_index[0] + 1
    dst = edge_index[1] + 1
    node_ids = jnp.arange(1, N + 1, dtype=jnp.int32)
    zeros_ids = jnp.zeros((N,), jnp.int32)
    src_full = jnp.concatenate([src, zeros_ids, node_ids])
    dst_full = jnp.concatenate([dst, node_ids, zeros_ids])
    pad = EP - EF
    src_g = jnp.concatenate([src_full, jnp.zeros((pad,), jnp.int32)])
    dst_g = jnp.concatenate([dst_full, jnp.zeros((pad,), jnp.int32)])
    dst_s = jnp.concatenate(
        [dst_full, jnp.full((pad,), TRASH, jnp.int32)]).reshape(CHQ, 128)
    sdc = jnp.stack([src_g.reshape(NW, GIT, GC),
                     dst_g.reshape(NW, GIT, GC)], axis=2)

    ef5 = jnp.concatenate(
        [jnp.concatenate([edge_features,
                          jnp.ones((E, 1), f32),
                          jnp.zeros((E, 4), f32)], axis=1),
         jnp.tile(jnp.array([[0, 0, 0, 0, 1, 0, 0, 0]], f32), (2 * N, 1)),
         jnp.zeros((pad, 8), f32)], axis=0)

    hm = np.zeros((H, 4), np.float32)
    for h in range(HEADS):
        hm[h * DH:(h + 1) * DH, h] = 1.0
    hm8 = np.zeros((32, 4), np.float32)
    for h in range(HEADS):
        hm8[h * 8:h * 8 + 8, h] = 1.0
    hm = jnp.asarray(hm)
    hm8 = jnp.asarray(hm8)
    pm = jnp.asarray(hm.T)
    zrows = jnp.zeros((RPT, 24), f32)
    folded = _fold_weights(params)

    g_tok = graph_features @ params['graph_proj'][0] + params['graph_proj'][1]
    nfp = jnp.zeros((NP, 16), f32).at[:N, :9].set(node_features)
    xn = _nodeproj(nfp, jnp.zeros((16, H), f32).at[:9].set(
        params['node_proj'][0]), params['node_proj'][1])
    xp = jnp.concatenate([g_tok[None, :], xn[:NP - 1]], axis=0)

    for i, fw in enumerate(folded):
        qx, kv, sk = _node_stage(xp, fw['wqx'], fw['bqx'], fw['wkv'],
                                 fw['bkv'], fw['wsk'], fw['bsk'])
        kvg, qxg = _sc_gather(kv, qx, sdc)
        pv = _edge_stage(kvg, qxg, ef5, fw['m5'], hm, hm8, pm)
        agg4 = _sc_scatter(pv, dst_s, zrows)
        xp = _combine(agg4, sk, xp, fw['g'], fw['b'],
                      gelu=(i < NUM_LAYERS - 1))
    return xp[:N + 1]
